# baseline (speedup 1.0000x reference)
import functools

import jax
import jax.numpy as jnp
from jax import lax
from jax.experimental import pallas as pl
from jax.experimental.pallas import tpu as pltpu
from jax.experimental.pallas import tpu_sc as plsc

_NC = 2
_NS = 16
_NW = _NC * _NS
_LANES = 16
_CHUNK = 32768


def _sc_body(x_hbm, emb_hbm, out_hbm, xv, ev):
    c = lax.axis_index("c")
    s = lax.axis_index("s")
    wid = s * _NC + c
    work = x_hbm.shape[0] // _NW
    period = emb_hbm.shape[0]
    base = wid * work
    ebase = lax.rem(base, period)
    nchunks = work // _CHUNK

    def chunk_body(i, carry):
        off = i * _CHUNK
        pltpu.sync_copy(x_hbm.at[pl.ds(base + off, _CHUNK)], xv)
        pltpu.sync_copy(emb_hbm.at[pl.ds(ebase + off, _CHUNK)], ev)

        def add_body(j, acc):
            o = j * _LANES
            plsc.addupdate(xv.at[pl.ds(o, _LANES)], ev[pl.ds(o, _LANES)])
            return acc

        lax.fori_loop(0, _CHUNK // _LANES, add_body, 0)
        pltpu.sync_copy(xv, out_hbm.at[pl.ds(base + off, _CHUNK)])
        return carry

    lax.fori_loop(0, nchunks, chunk_body, 0)


def kernel(x, emb_weight):
    batch, seq_len, d_model = x.shape
    n = batch * seq_len * d_model
    xf = x.reshape(n)
    ef = emb_weight[:seq_len].reshape(seq_len * d_model)
    mesh = plsc.VectorSubcoreMesh(core_axis_name="c", subcore_axis_name="s")
    k = functools.partial(
        pl.kernel,
        mesh=mesh,
        out_type=jax.ShapeDtypeStruct((n,), x.dtype),
        scratch_types=[
            pltpu.VMEM((_CHUNK,), jnp.float32),
            pltpu.VMEM((_CHUNK,), jnp.float32),
        ],
    )(_sc_body)
    return k(xf, ef).reshape(batch, seq_len, d_model)


# speedup vs baseline: 1.5976x; 1.5976x over previous
import functools

import jax
import jax.numpy as jnp
from jax import lax
from jax.experimental import pallas as pl
from jax.experimental.pallas import tpu as pltpu
from jax.experimental.pallas import tpu_sc as plsc

_NC = 2
_NS = 16
_NW = _NC * _NS
_LANES = 16
_CHUNK = 16384
_UNROLL = 8


def _sc_body(x_hbm, emb_hbm, out_hbm, xv0, xv1, ev0, ev1, sin0, sin1, sout0, sout1):
    c = lax.axis_index("c")
    s = lax.axis_index("s")
    wid = s * _NC + c
    work = x_hbm.shape[0] // _NW
    period = emb_hbm.shape[0]
    base = wid * work
    ebase = lax.rem(base, period)
    nchunks = work // _CHUNK
    xv = (xv0, xv1)
    ev = (ev0, ev1)
    sin = (sin0, sin1)
    sout = (sout0, sout1)

    def start_in(ch, p):
        off = ch * _CHUNK
        pltpu.make_async_copy(x_hbm.at[pl.ds(base + off, _CHUNK)], xv[p], sin[p]).start()
        pltpu.make_async_copy(emb_hbm.at[pl.ds(ebase + off, _CHUNK)], ev[p], sin[p]).start()

    def wait_in(p):
        pltpu.make_async_copy(x_hbm.at[pl.ds(base, _CHUNK)], xv[p], sin[p]).wait()
        pltpu.make_async_copy(emb_hbm.at[pl.ds(ebase, _CHUNK)], ev[p], sin[p]).wait()

    def start_out(ch, p):
        off = ch * _CHUNK
        pltpu.make_async_copy(xv[p], out_hbm.at[pl.ds(base + off, _CHUNK)], sout[p]).start()

    def wait_out(p):
        pltpu.make_async_copy(xv[p], out_hbm.at[pl.ds(base, _CHUNK)], sout[p]).wait()

    start_in(0, 0)

    def body(i, carry):
        for p in (0, 1):
            ch = 2 * i + p
            q = 1 - p

            @pl.when(jnp.logical_and(ch + 1 < nchunks, ch >= 1))
            def _():
                wait_out(q)

            @pl.when(ch + 1 < nchunks)
            def _():
                start_in(ch + 1, q)

            wait_in(p)

            def add_body(j, acc):
                b0 = j * (_LANES * _UNROLL)
                for u in range(_UNROLL):
                    o = b0 + u * _LANES
                    plsc.addupdate(xv[p].at[pl.ds(o, _LANES)], ev[p][pl.ds(o, _LANES)])
                return acc

            lax.fori_loop(0, _CHUNK // (_LANES * _UNROLL), add_body, 0)
            start_out(ch, p)
        return carry

    lax.fori_loop(0, nchunks // 2, body, 0)
    wait_out(0)
    wait_out(1)


def kernel(x, emb_weight):
    batch, seq_len, d_model = x.shape
    n = batch * seq_len * d_model
    xf = x.reshape(n)
    ef = emb_weight[:seq_len].reshape(seq_len * d_model)
    mesh = plsc.VectorSubcoreMesh(core_axis_name="c", subcore_axis_name="s")
    k = functools.partial(
        pl.kernel,
        mesh=mesh,
        out_type=jax.ShapeDtypeStruct((n,), x.dtype),
        scratch_types=[
            pltpu.VMEM((_CHUNK,), jnp.float32),
            pltpu.VMEM((_CHUNK,), jnp.float32),
            pltpu.VMEM((_CHUNK,), jnp.float32),
            pltpu.VMEM((_CHUNK,), jnp.float32),
            pltpu.SemaphoreType.DMA,
            pltpu.SemaphoreType.DMA,
            pltpu.SemaphoreType.DMA,
            pltpu.SemaphoreType.DMA,
        ],
    )(_sc_body)
    return k(xf, ef).reshape(batch, seq_len, d_model)
